# Initial kernel scaffold; baseline (speedup 1.0000x reference)
import jax
import jax.numpy as jnp
from jax.experimental import pallas as pl


def kernel(dst, ref):
    raise NotImplementedError("write your pallas kernel here")


# speedup vs baseline: 257.3916x; 257.3916x over previous
import dataclasses
import functools

import numpy as np
import jax
import jax.numpy as jnp
from jax import lax
from jax.experimental import pallas as pl
from jax.experimental.pallas import tpu as pltpu
from jax.experimental.pallas import tpu_sc as plsc

_NPLANE = 48
_NPIX = 262144
_HALF = _NPIX // 2
_NUNIT = 24
_NTASK = 96
_NWORKER = 32


def _unit_row(k):
    return k if k < 16 else 2 * k - 16


_HBLK = 32768
_ABLK = 16384


def _sc_compiler_params():
    cp = pltpu.CompilerParams()
    if "needs_layout_passes" in pltpu.CompilerParams.__dataclass_fields__:
        cp = dataclasses.replace(cp, needs_layout_passes=False)
    return cp


def _hist_sc(dflat, rflat):
    mesh = plsc.VectorSubcoreMesh(core_axis_name="c", subcore_axis_name="s")
    nhb = _HALF // _HBLK

    @functools.partial(
        pl.kernel,
        mesh=mesh,
        compiler_params=_sc_compiler_params(),
        out_type=jax.ShapeDtypeStruct((_NTASK * 256,), jnp.float32),
        scratch_types=[
            pltpu.VMEM((_HBLK,), jnp.float32),
            pltpu.VMEM((_HBLK,), jnp.float32),
            pltpu.VMEM((4096,), jnp.float32),
            pltpu.VMEM((256,), jnp.float32),
            pltpu.SemaphoreType.DMA,
            pltpu.SemaphoreType.DMA,
        ],
    )
    def hist_kernel(d_hbm, r_hbm, out_hbm, buf0, buf1, h16, hist, sem0, sem1):
        wid = lax.axis_index("s") * 2 + lax.axis_index("c")
        zeros16 = jnp.zeros((16,), jnp.float32)
        ones16 = jnp.full((16,), 1.0, jnp.float32)
        laneoff = lax.iota(jnp.int32, 16) * 256

        def bin_block(buf):
            @pl.loop(0, _HBLK, step=128)
            def _(c0):
                for jj in range(8):
                    x = buf[pl.ds(c0 + jj * 16, 16)]
                    i = (x * 256.0).astype(jnp.int32)
                    i = jnp.minimum(jnp.maximum(i, 0), 255)
                    plsc.addupdate_scatter(h16, [i + laneoff], ones16)

        def do_task(src, base, t):
            @pl.loop(0, 4096, step=16)
            def _(i0):
                h16[pl.ds(i0, 16)] = zeros16

            cp0 = pltpu.async_copy(src.at[pl.ds(base, _HBLK)], buf0, sem0)
            cp1 = pltpu.async_copy(
                src.at[pl.ds(base + _HBLK, _HBLK)], buf1, sem1)
            cp0.wait()
            bin_block(buf0)
            cp2 = pltpu.async_copy(
                src.at[pl.ds(base + 2 * _HBLK, _HBLK)], buf0, sem0)
            cp1.wait()
            bin_block(buf1)
            cp3 = pltpu.async_copy(
                src.at[pl.ds(base + 3 * _HBLK, _HBLK)], buf1, sem1)
            cp2.wait()
            bin_block(buf0)
            cp3.wait()
            bin_block(buf1)

            for c in range(16):
                acc = h16[pl.ds(c * 16, 16)]
                for l in range(1, 16):
                    acc = acc + h16[pl.ds(l * 256 + c * 16, 16)]
                hist[pl.ds(c * 16, 16)] = acc

            pltpu.sync_copy(hist, out_hbm.at[pl.ds(t * 256, 256)])

        for r in range(3):
            t = wid * 3 + r
            u = t % 48
            h = t // 48
            img = u // 24
            k = u % 24
            q = jnp.where(k < 16, k, 2 * k - 16)
            base = q * _NPIX + h * _HALF

            @pl.when(img == 0)
            def _():
                do_task(d_hbm, base, t)

            @pl.when(img == 1)
            def _():
                do_task(r_hbm, base, t)

    return hist_kernel(dflat, rflat)


def _tables_tc(partials):
    def body(p_ref, o_ref):
        p = p_ref[...]
        hsum = p[:48] + p[48:]
        norm = jnp.maximum(
            jnp.sum(jnp.abs(hsum), axis=1, keepdims=True), 1e-12)
        x = hsum / norm
        for s in (1, 2, 4, 8, 16, 32, 64, 128):
            x = x + jnp.concatenate(
                [jnp.zeros((48, s), jnp.float32), x[:, :256 - s]], axis=1)
        hd = x[:24]
        hr = x[24:]
        runaway = hd[:, 1:2] < hr[:, 0:1]
        for c in range(16):
            hdc = hd[:, c * 16:(c + 1) * 16]
            lt = (hr[:, None, :] < hdc[:, :, None]).astype(jnp.float32)
            jmask = (lax.broadcasted_iota(jnp.int32, (24, 16, 256), 2)
                     >= 1).astype(jnp.float32)
            cnt = jnp.sum(lt * jmask, axis=2)
            tab = jnp.minimum(cnt + 1.0, 255.0)
            tab = jnp.where(runaway, 255.0, tab)
            ii = lax.broadcasted_iota(jnp.int32, (24, 16), 1) + c * 16
            tab = jnp.where(ii == 0, 0.0, tab)
            tab = jnp.where(ii == 255, 255.0, tab)
            o_ref[:, c * 16:(c + 1) * 16] = tab / 255.0

    return pl.pallas_call(
        body,
        out_shape=jax.ShapeDtypeStruct((_NUNIT, 256), jnp.float32),
    )(partials)


def _apply_sc(dflat, luts_flat):
    mesh = plsc.VectorSubcoreMesh(core_axis_name="c", subcore_axis_name="s")
    nab = _HALF // _ABLK

    @functools.partial(
        pl.kernel,
        mesh=mesh,
        compiler_params=_sc_compiler_params(),
        out_type=jax.ShapeDtypeStruct((_NPLANE * _NPIX,), jnp.float32),
        scratch_types=[
            pltpu.VMEM((_ABLK,), jnp.float32),
            pltpu.VMEM((_ABLK,), jnp.float32),
            pltpu.VMEM((_ABLK,), jnp.float32),
            pltpu.VMEM((_ABLK,), jnp.float32),
            pltpu.VMEM((256,), jnp.float32),
            pltpu.SemaphoreType.DMA,
            pltpu.SemaphoreType.DMA,
            pltpu.SemaphoreType.DMA,
            pltpu.SemaphoreType.DMA,
        ],
    )
    def apply_kernel(d_hbm, lut_hbm, out_hbm, ibuf0, ibuf1, obuf0, obuf1,
                     lut, isem0, isem1, osem0, osem1):
        wid = lax.axis_index("s") * 2 + lax.axis_index("c")
        ibufs = (ibuf0, ibuf1)
        obufs = (obuf0, obuf1)
        isems = (isem0, isem1)
        osems = (osem0, osem1)

        def lut_block(ib, ob):
            @pl.loop(0, _ABLK, step=128)
            def _(c0):
                for jj in range(8):
                    sl = pl.ds(c0 + jj * 16, 16)
                    x = ib[sl]
                    i = (x * 255.0).astype(jnp.int32)
                    i = jnp.minimum(i, 255)
                    ob[sl] = plsc.load_gather(lut, [i])

        for r in range(3):
            t = wid * 3 + r
            p = t // 2
            h = t % 2
            base = p * _NPIX + h * _HALF
            pltpu.sync_copy(lut_hbm.at[pl.ds(p * 256, 256)], lut)

            icps = [None] * nab
            ocps = [None] * nab
            icps[0] = pltpu.async_copy(
                d_hbm.at[pl.ds(base, _ABLK)], ibuf0, isem0)
            icps[1] = pltpu.async_copy(
                d_hbm.at[pl.ds(base + _ABLK, _ABLK)], ibuf1, isem1)
            for ib in range(nab):
                bi = ib % 2
                icps[ib].wait()
                if ib >= 2:
                    ocps[ib - 2].wait()
                lut_block(ibufs[bi], obufs[bi])
                ocps[ib] = pltpu.async_copy(
                    obufs[bi], out_hbm.at[pl.ds(base + ib * _ABLK, _ABLK)],
                    osems[bi])
                if ib + 2 < nab:
                    icps[ib + 2] = pltpu.async_copy(
                        d_hbm.at[pl.ds(base + (ib + 2) * _ABLK, _ABLK)],
                        ibufs[bi], isems[bi])
            ocps[nab - 2].wait()
            ocps[nab - 1].wait()

    return apply_kernel(dflat, luts_flat)


_KMAP = np.array(
    [(lambda q: q if q < 16 else (q + 16) // 2)((p // 3) * (p % 3))
     for p in range(_NPLANE)],
    dtype=np.int32,
)


@jax.jit
def kernel(dst, ref):
    b, c, h, w = dst.shape
    dflat = dst.reshape(-1)
    rflat = ref.reshape(-1)
    partials = _hist_sc(dflat, rflat).reshape(_NTASK, 256)
    luts24 = _tables_tc(partials)
    luts48 = luts24[_KMAP].reshape(-1)
    out = _apply_sc(dflat, luts48)
    return out.reshape(b, c, h, w)


# speedup vs baseline: 268.0449x; 1.0414x over previous
import dataclasses
import functools

import numpy as np
import jax
import jax.numpy as jnp
from jax import lax
from jax.experimental import pallas as pl
from jax.experimental.pallas import tpu as pltpu
from jax.experimental.pallas import tpu_sc as plsc

_NPLANE = 48
_NPIX = 262144
_HALF = _NPIX // 2
_NUNIT = 24
_NTASK = 96
_NWORKER = 32


def _unit_row(k):
    return k if k < 16 else 2 * k - 16


_HBLK = 32768
_ABLK = 16384


def _sc_compiler_params():
    cp = pltpu.CompilerParams()
    if "needs_layout_passes" in pltpu.CompilerParams.__dataclass_fields__:
        cp = dataclasses.replace(cp, needs_layout_passes=False)
    return cp


def _hist_sc(dflat, rflat):
    mesh = plsc.VectorSubcoreMesh(core_axis_name="c", subcore_axis_name="s")
    nhb = _HALF // _HBLK

    @functools.partial(
        pl.kernel,
        mesh=mesh,
        compiler_params=_sc_compiler_params(),
        out_type=jax.ShapeDtypeStruct((_NTASK * 256,), jnp.float32),
        scratch_types=[
            pltpu.VMEM((_HBLK,), jnp.float32),
            pltpu.VMEM((_HBLK,), jnp.float32),
            pltpu.VMEM((4096,), jnp.float32),
            pltpu.VMEM((256,), jnp.float32),
            pltpu.SemaphoreType.DMA,
            pltpu.SemaphoreType.DMA,
        ],
    )
    def hist_kernel(d_hbm, r_hbm, out_hbm, buf0, buf1, h16, hist, sem0, sem1):
        wid = lax.axis_index("s") * 2 + lax.axis_index("c")
        zeros16 = jnp.zeros((16,), jnp.float32)
        ones16 = jnp.full((16,), 1.0, jnp.float32)
        lane = lax.iota(jnp.int32, 16)

        def bin_block(buf):
            @pl.loop(0, _HBLK, step=128)
            def _(c0):
                for jj in range(8):
                    x = buf[pl.ds(c0 + jj * 16, 16)]
                    i = (x * 256.0).astype(jnp.int32)
                    plsc.addupdate_scatter(h16, [i * 16 + lane], ones16)

        def do_task(src, base, t):
            @pl.loop(0, 4096, step=16)
            def _(i0):
                h16[pl.ds(i0, 16)] = zeros16

            cp0 = pltpu.async_copy(src.at[pl.ds(base, _HBLK)], buf0, sem0)
            cp1 = pltpu.async_copy(
                src.at[pl.ds(base + _HBLK, _HBLK)], buf1, sem1)
            cp0.wait()
            bin_block(buf0)
            cp2 = pltpu.async_copy(
                src.at[pl.ds(base + 2 * _HBLK, _HBLK)], buf0, sem0)
            cp1.wait()
            bin_block(buf1)
            cp3 = pltpu.async_copy(
                src.at[pl.ds(base + 3 * _HBLK, _HBLK)], buf1, sem1)
            cp2.wait()
            bin_block(buf0)
            cp3.wait()
            bin_block(buf1)

            for c in range(16):
                acc = plsc.load_gather(h16, [lane * 16 + c * 256])
                for l in range(1, 16):
                    acc = acc + plsc.load_gather(h16, [lane * 16 + (c * 256 + l)])
                hist[pl.ds(c * 16, 16)] = acc

            pltpu.sync_copy(hist, out_hbm.at[pl.ds(t * 256, 256)])

        for r in range(3):
            t = wid * 3 + r
            u = t % 48
            h = t // 48
            img = u // 24
            k = u % 24
            q = jnp.where(k < 16, k, 2 * k - 16)
            base = q * _NPIX + h * _HALF

            @pl.when(img == 0)
            def _():
                do_task(d_hbm, base, t)

            @pl.when(img == 1)
            def _():
                do_task(r_hbm, base, t)

    return hist_kernel(dflat, rflat)


def _tables_tc(partials):
    def body(p_ref, o_ref):
        p = p_ref[...]
        hsum = p[:48] + p[48:]
        norm = jnp.maximum(
            jnp.sum(jnp.abs(hsum), axis=1, keepdims=True), 1e-12)
        x = hsum / norm
        for s in (1, 2, 4, 8, 16, 32, 64, 128):
            x = x + jnp.concatenate(
                [jnp.zeros((48, s), jnp.float32), x[:, :256 - s]], axis=1)
        hd = x[:24]
        hr = x[24:]
        runaway = hd[:, 1:2] < hr[:, 0:1]
        for c in range(16):
            hdc = hd[:, c * 16:(c + 1) * 16]
            lt = (hr[:, None, :] < hdc[:, :, None]).astype(jnp.float32)
            jmask = (lax.broadcasted_iota(jnp.int32, (24, 16, 256), 2)
                     >= 1).astype(jnp.float32)
            cnt = jnp.sum(lt * jmask, axis=2)
            tab = jnp.minimum(cnt + 1.0, 255.0)
            tab = jnp.where(runaway, 255.0, tab)
            ii = lax.broadcasted_iota(jnp.int32, (24, 16), 1) + c * 16
            tab = jnp.where(ii == 0, 0.0, tab)
            tab = jnp.where(ii == 255, 255.0, tab)
            lutc = tab / 255.0
            o_ref[:, c * 16:(c + 1) * 16, :] = jnp.broadcast_to(
                lutc[:, :, None], (_NUNIT, 16, 16))

    return pl.pallas_call(
        body,
        out_shape=jax.ShapeDtypeStruct((_NUNIT, 256, 16), jnp.float32),
    )(partials)


def _apply_sc(dflat, luts_flat):
    mesh = plsc.VectorSubcoreMesh(core_axis_name="c", subcore_axis_name="s")
    nab = _HALF // _ABLK

    @functools.partial(
        pl.kernel,
        mesh=mesh,
        compiler_params=_sc_compiler_params(),
        out_type=jax.ShapeDtypeStruct((_NPLANE * _NPIX,), jnp.float32),
        scratch_types=[
            pltpu.VMEM((_ABLK,), jnp.float32),
            pltpu.VMEM((_ABLK,), jnp.float32),
            pltpu.VMEM((_ABLK,), jnp.float32),
            pltpu.VMEM((_ABLK,), jnp.float32),
            pltpu.VMEM((4096,), jnp.float32),
            pltpu.SemaphoreType.DMA,
            pltpu.SemaphoreType.DMA,
            pltpu.SemaphoreType.DMA,
            pltpu.SemaphoreType.DMA,
        ],
    )
    def apply_kernel(d_hbm, lut_hbm, out_hbm, ibuf0, ibuf1, obuf0, obuf1,
                     lut, isem0, isem1, osem0, osem1):
        wid = lax.axis_index("s") * 2 + lax.axis_index("c")
        lane = lax.iota(jnp.int32, 16)
        ibufs = (ibuf0, ibuf1)
        obufs = (obuf0, obuf1)
        isems = (isem0, isem1)
        osems = (osem0, osem1)

        def lut_block(ib, ob):
            @pl.loop(0, _ABLK, step=128)
            def _(c0):
                for jj in range(8):
                    sl = pl.ds(c0 + jj * 16, 16)
                    x = ib[sl]
                    i = (x * 255.0).astype(jnp.int32)
                    ob[sl] = plsc.load_gather(lut, [i * 16 + lane])

        for r in range(3):
            t = wid * 3 + r
            p = t // 2
            h = t % 2
            base = p * _NPIX + h * _HALF
            pltpu.sync_copy(lut_hbm.at[pl.ds(p * 4096, 4096)], lut)

            icps = [None] * nab
            ocps = [None] * nab
            icps[0] = pltpu.async_copy(
                d_hbm.at[pl.ds(base, _ABLK)], ibuf0, isem0)
            icps[1] = pltpu.async_copy(
                d_hbm.at[pl.ds(base + _ABLK, _ABLK)], ibuf1, isem1)
            for ib in range(nab):
                bi = ib % 2
                icps[ib].wait()
                if ib >= 2:
                    ocps[ib - 2].wait()
                lut_block(ibufs[bi], obufs[bi])
                ocps[ib] = pltpu.async_copy(
                    obufs[bi], out_hbm.at[pl.ds(base + ib * _ABLK, _ABLK)],
                    osems[bi])
                if ib + 2 < nab:
                    icps[ib + 2] = pltpu.async_copy(
                        d_hbm.at[pl.ds(base + (ib + 2) * _ABLK, _ABLK)],
                        ibufs[bi], isems[bi])
            ocps[nab - 2].wait()
            ocps[nab - 1].wait()

    return apply_kernel(dflat, luts_flat)


_KMAP = np.array(
    [(lambda q: q if q < 16 else (q + 16) // 2)((p // 3) * (p % 3))
     for p in range(_NPLANE)],
    dtype=np.int32,
)


@jax.jit
def kernel(dst, ref):
    b, c, h, w = dst.shape
    dflat = dst.reshape(-1)
    rflat = ref.reshape(-1)
    partials = _hist_sc(dflat, rflat).reshape(_NTASK, 256)
    luts24 = _tables_tc(partials)
    luts48 = luts24[_KMAP].reshape(-1)
    out = _apply_sc(dflat, luts48)
    return out.reshape(b, c, h, w)


# speedup vs baseline: 578.6708x; 2.1589x over previous
import dataclasses
import functools

import numpy as np
import jax
import jax.numpy as jnp
from jax import lax
from jax.experimental import pallas as pl
from jax.experimental.pallas import tpu as pltpu
from jax.experimental.pallas import tpu_sc as plsc

_NPLANE = 48
_NPIX = 262144
_HALF = _NPIX // 2
_NUNIT = 24
_NTASK = 96
_NWORKER = 32


def _unit_row(k):
    return k if k < 16 else 2 * k - 16


_HBLK = 32768
_ABLK = 16384


def _sc_compiler_params():
    cp = pltpu.CompilerParams()
    if "needs_layout_passes" in pltpu.CompilerParams.__dataclass_fields__:
        cp = dataclasses.replace(cp, needs_layout_passes=False)
    return cp


def _hist_sc(dflat, rflat):
    mesh = plsc.VectorSubcoreMesh(core_axis_name="c", subcore_axis_name="s")
    nhb = _HALF // _HBLK

    @functools.partial(
        pl.kernel,
        mesh=mesh,
        compiler_params=_sc_compiler_params(),
        out_type=jax.ShapeDtypeStruct((_NTASK * 256,), jnp.float32),
        scratch_types=[
            pltpu.VMEM((_HBLK,), jnp.float32),
            pltpu.VMEM((_HBLK,), jnp.float32),
            pltpu.VMEM((4096,), jnp.float32),
            pltpu.VMEM((256,), jnp.float32),
            pltpu.SemaphoreType.DMA,
            pltpu.SemaphoreType.DMA,
        ],
    )
    def hist_kernel(d_hbm, r_hbm, out_hbm, buf0, buf1, h16, hist, sem0, sem1):
        wid = lax.axis_index("s") * 2 + lax.axis_index("c")
        zeros16 = jnp.zeros((16,), jnp.float32)
        ones16 = jnp.full((16,), 1.0, jnp.float32)
        lane = lax.iota(jnp.int32, 16)

        def bin_block(buf):
            @plsc.parallel_loop(0, _HBLK, step=16, unroll=8)
            def _(c0):
                x = buf[pl.ds(c0, 16)]
                i = (x * 256.0).astype(jnp.int32)
                plsc.addupdate_scatter(h16, [i * 16 + lane], ones16)

        def do_task(src, base, t):
            @pl.loop(0, 4096, step=16)
            def _(i0):
                h16[pl.ds(i0, 16)] = zeros16

            cp0 = pltpu.async_copy(src.at[pl.ds(base, _HBLK)], buf0, sem0)
            cp1 = pltpu.async_copy(
                src.at[pl.ds(base + _HBLK, _HBLK)], buf1, sem1)
            cp0.wait()
            bin_block(buf0)
            cp2 = pltpu.async_copy(
                src.at[pl.ds(base + 2 * _HBLK, _HBLK)], buf0, sem0)
            cp1.wait()
            bin_block(buf1)
            cp3 = pltpu.async_copy(
                src.at[pl.ds(base + 3 * _HBLK, _HBLK)], buf1, sem1)
            cp2.wait()
            bin_block(buf0)
            cp3.wait()
            bin_block(buf1)

            for c in range(16):
                acc = plsc.load_gather(h16, [lane * 16 + c * 256])
                for l in range(1, 16):
                    acc = acc + plsc.load_gather(h16, [lane * 16 + (c * 256 + l)])
                hist[pl.ds(c * 16, 16)] = acc

            pltpu.sync_copy(hist, out_hbm.at[pl.ds(t * 256, 256)])

        for r in range(3):
            t = wid * 3 + r
            u = t % 48
            h = t // 48
            img = u // 24
            k = u % 24
            q = jnp.where(k < 16, k, 2 * k - 16)
            base = q * _NPIX + h * _HALF

            @pl.when(img == 0)
            def _():
                do_task(d_hbm, base, t)

            @pl.when(img == 1)
            def _():
                do_task(r_hbm, base, t)

    return hist_kernel(dflat, rflat)


def _tables_tc(partials):
    def body(p_ref, o_ref):
        p = p_ref[...]
        hsum = p[:48] + p[48:]
        norm = jnp.maximum(
            jnp.sum(jnp.abs(hsum), axis=1, keepdims=True), 1e-12)
        x = hsum / norm
        for s in (1, 2, 4, 8, 16, 32, 64, 128):
            x = x + jnp.concatenate(
                [jnp.zeros((48, s), jnp.float32), x[:, :256 - s]], axis=1)
        hd = x[:24]
        hr = x[24:]
        runaway = hd[:, 1:2] < hr[:, 0:1]
        for c in range(16):
            hdc = hd[:, c * 16:(c + 1) * 16]
            lt = (hr[:, None, :] < hdc[:, :, None]).astype(jnp.float32)
            jmask = (lax.broadcasted_iota(jnp.int32, (24, 16, 256), 2)
                     >= 1).astype(jnp.float32)
            cnt = jnp.sum(lt * jmask, axis=2)
            tab = jnp.minimum(cnt + 1.0, 255.0)
            tab = jnp.where(runaway, 255.0, tab)
            ii = lax.broadcasted_iota(jnp.int32, (24, 16), 1) + c * 16
            tab = jnp.where(ii == 0, 0.0, tab)
            tab = jnp.where(ii == 255, 255.0, tab)
            lutc = tab / 255.0
            o_ref[:, c * 16:(c + 1) * 16, :] = jnp.broadcast_to(
                lutc[:, :, None], (_NUNIT, 16, 16))

    return pl.pallas_call(
        body,
        out_shape=jax.ShapeDtypeStruct((_NUNIT, 256, 16), jnp.float32),
    )(partials)


def _apply_sc(dflat, luts_flat):
    mesh = plsc.VectorSubcoreMesh(core_axis_name="c", subcore_axis_name="s")
    nab = _HALF // _ABLK

    @functools.partial(
        pl.kernel,
        mesh=mesh,
        compiler_params=_sc_compiler_params(),
        out_type=jax.ShapeDtypeStruct((_NPLANE * _NPIX,), jnp.float32),
        scratch_types=[
            pltpu.VMEM((_ABLK,), jnp.float32),
            pltpu.VMEM((_ABLK,), jnp.float32),
            pltpu.VMEM((_ABLK,), jnp.float32),
            pltpu.VMEM((_ABLK,), jnp.float32),
            pltpu.VMEM((4096,), jnp.float32),
            pltpu.SemaphoreType.DMA,
            pltpu.SemaphoreType.DMA,
            pltpu.SemaphoreType.DMA,
            pltpu.SemaphoreType.DMA,
        ],
    )
    def apply_kernel(d_hbm, lut_hbm, out_hbm, ibuf0, ibuf1, obuf0, obuf1,
                     lut, isem0, isem1, osem0, osem1):
        wid = lax.axis_index("s") * 2 + lax.axis_index("c")
        lane = lax.iota(jnp.int32, 16)
        ibufs = (ibuf0, ibuf1)
        obufs = (obuf0, obuf1)
        isems = (isem0, isem1)
        osems = (osem0, osem1)

        def lut_block(ib, ob):
            @plsc.parallel_loop(0, _ABLK, step=16, unroll=8)
            def _(c0):
                sl = pl.ds(c0, 16)
                x = ib[sl]
                i = (x * 255.0).astype(jnp.int32)
                ob[sl] = plsc.load_gather(lut, [i * 16 + lane])

        for r in range(3):
            t = wid * 3 + r
            p = t // 2
            h = t % 2
            base = p * _NPIX + h * _HALF
            pltpu.sync_copy(lut_hbm.at[pl.ds(p * 4096, 4096)], lut)

            icps = [None] * nab
            ocps = [None] * nab
            icps[0] = pltpu.async_copy(
                d_hbm.at[pl.ds(base, _ABLK)], ibuf0, isem0)
            icps[1] = pltpu.async_copy(
                d_hbm.at[pl.ds(base + _ABLK, _ABLK)], ibuf1, isem1)
            for ib in range(nab):
                bi = ib % 2
                icps[ib].wait()
                if ib >= 2:
                    ocps[ib - 2].wait()
                lut_block(ibufs[bi], obufs[bi])
                ocps[ib] = pltpu.async_copy(
                    obufs[bi], out_hbm.at[pl.ds(base + ib * _ABLK, _ABLK)],
                    osems[bi])
                if ib + 2 < nab:
                    icps[ib + 2] = pltpu.async_copy(
                        d_hbm.at[pl.ds(base + (ib + 2) * _ABLK, _ABLK)],
                        ibufs[bi], isems[bi])
            ocps[nab - 2].wait()
            ocps[nab - 1].wait()

    return apply_kernel(dflat, luts_flat)


_KMAP = np.array(
    [(lambda q: q if q < 16 else (q + 16) // 2)((p // 3) * (p % 3))
     for p in range(_NPLANE)],
    dtype=np.int32,
)


@jax.jit
def kernel(dst, ref):
    b, c, h, w = dst.shape
    dflat = dst.reshape(-1)
    rflat = ref.reshape(-1)
    partials = _hist_sc(dflat, rflat).reshape(_NTASK, 256)
    luts24 = _tables_tc(partials)
    luts48 = luts24[_KMAP].reshape(-1)
    out = _apply_sc(dflat, luts48)
    return out.reshape(b, c, h, w)


# speedup vs baseline: 1104.0221x; 1.9079x over previous
import dataclasses
import functools

import numpy as np
import jax
import jax.numpy as jnp
from jax import lax
from jax.experimental import pallas as pl
from jax.experimental.pallas import tpu as pltpu
from jax.experimental.pallas import tpu_sc as plsc

_NPLANE = 48
_NPIX = 262144
_NUNIT = 24
_NTASK = 96
_W = 512

_HROWS = 64
_AROWS = 32


def _sc_compiler_params():
    cp = pltpu.CompilerParams()
    if "needs_layout_passes" in pltpu.CompilerParams.__dataclass_fields__:
        cp = dataclasses.replace(cp, needs_layout_passes=False)
    return cp


def _vec_loop(buf, nrows, unroll, fn):
    @plsc.parallel_loop(0, nrows * (_W // 16), step=1, unroll=unroll)
    def _(v):
        sl = (v // (_W // 16), pl.ds((v % (_W // 16)) * 16, 16))
        fn(buf[sl], sl)


def _hist_sc(dst, ref):
    mesh = plsc.VectorSubcoreMesh(core_axis_name="c", subcore_axis_name="s")

    @functools.partial(
        pl.kernel,
        mesh=mesh,
        compiler_params=_sc_compiler_params(),
        out_type=jax.ShapeDtypeStruct((_NTASK * 256,), jnp.float32),
        scratch_types=[
            pltpu.VMEM((_HROWS, _W), jnp.float32),
            pltpu.VMEM((_HROWS, _W), jnp.float32),
            pltpu.VMEM((4096,), jnp.float32),
            pltpu.VMEM((256,), jnp.float32),
            pltpu.SemaphoreType.DMA,
            pltpu.SemaphoreType.DMA,
        ],
    )
    def hist_kernel(d_hbm, r_hbm, out_hbm, buf0, buf1, h16, hist, sem0, sem1):
        wid = lax.axis_index("s") * 2 + lax.axis_index("c")
        zeros16 = jnp.zeros((16,), jnp.float32)
        ones16 = jnp.full((16,), 1.0, jnp.float32)
        lane = lax.iota(jnp.int32, 16)

        def bin_vec(x, sl):
            i = (x * 256.0).astype(jnp.int32)
            plsc.addupdate_scatter(h16, [i * 16 + lane], ones16)

        def bin_block(buf):
            _vec_loop(buf, _HROWS, 8, bin_vec)

        def do_task(src, bb, cc, r00, t):
            @pl.loop(0, 4096, step=16)
            def _(i0):
                h16[pl.ds(i0, 16)] = zeros16

            def start(ib, buf, sem):
                return pltpu.async_copy(
                    src.at[bb, cc, pl.ds(r00 + ib * _HROWS, _HROWS), :],
                    buf, sem)

            cp0 = start(0, buf0, sem0)
            cp1 = start(1, buf1, sem1)
            cp0.wait()
            bin_block(buf0)
            cp2 = start(2, buf0, sem0)
            cp1.wait()
            bin_block(buf1)
            cp3 = start(3, buf1, sem1)
            cp2.wait()
            bin_block(buf0)
            cp3.wait()
            bin_block(buf1)

            for c in range(16):
                acc = plsc.load_gather(h16, [lane * 16 + c * 256])
                for l in range(1, 16):
                    acc = acc + plsc.load_gather(
                        h16, [lane * 16 + (c * 256 + l)])
                hist[pl.ds(c * 16, 16)] = acc

            pltpu.sync_copy(hist, out_hbm.at[pl.ds(t * 256, 256)])

        for r in range(3):
            t = wid * 3 + r
            u = t % 48
            h = t // 48
            img = u // 24
            k = u % 24
            q = jnp.where(k < 16, k, 2 * k - 16)
            bb = q // 3
            cc = q % 3
            r00 = h * 256

            @pl.when(img == 0)
            def _():
                do_task(d_hbm, bb, cc, r00, t)

            @pl.when(img == 1)
            def _():
                do_task(r_hbm, bb, cc, r00, t)

    return hist_kernel(dst, ref)


def _tables_tc(partials):
    def body(p_ref, o_ref):
        p = p_ref[...]
        hsum = p[:48] + p[48:]
        norm = jnp.maximum(
            jnp.sum(jnp.abs(hsum), axis=1, keepdims=True), 1e-12)
        x = hsum / norm
        for s in (1, 2, 4, 8, 16, 32, 64, 128):
            x = x + jnp.concatenate(
                [jnp.zeros((48, s), jnp.float32), x[:, :256 - s]], axis=1)
        hd = x[:24]
        hr = x[24:]
        runaway = hd[:, 1:2] < hr[:, 0:1]
        for c in range(16):
            hdc = hd[:, c * 16:(c + 1) * 16]
            lt = (hr[:, None, :] < hdc[:, :, None]).astype(jnp.float32)
            jmask = (lax.broadcasted_iota(jnp.int32, (24, 16, 256), 2)
                     >= 1).astype(jnp.float32)
            cnt = jnp.sum(lt * jmask, axis=2)
            tab = jnp.minimum(cnt + 1.0, 255.0)
            tab = jnp.where(runaway, 255.0, tab)
            ii = lax.broadcasted_iota(jnp.int32, (24, 16), 1) + c * 16
            tab = jnp.where(ii == 0, 0.0, tab)
            tab = jnp.where(ii == 255, 255.0, tab)
            lutc = tab / 255.0
            o_ref[:, c * 16:(c + 1) * 16, :] = jnp.broadcast_to(
                lutc[:, :, None], (_NUNIT, 16, 16))

    return pl.pallas_call(
        body,
        out_shape=jax.ShapeDtypeStruct((_NUNIT, 256, 16), jnp.float32),
    )(partials)


def _apply_sc(dst, luts_flat):
    mesh = plsc.VectorSubcoreMesh(core_axis_name="c", subcore_axis_name="s")
    nab = _NPIX // 2 // (_AROWS * _W)

    @functools.partial(
        pl.kernel,
        mesh=mesh,
        compiler_params=_sc_compiler_params(),
        out_type=jax.ShapeDtypeStruct((16, 3, 512, 512), jnp.float32),
        scratch_types=[
            pltpu.VMEM((_AROWS, _W), jnp.float32),
            pltpu.VMEM((_AROWS, _W), jnp.float32),
            pltpu.VMEM((_AROWS, _W), jnp.float32),
            pltpu.VMEM((_AROWS, _W), jnp.float32),
            pltpu.VMEM((4096,), jnp.float32),
            pltpu.SemaphoreType.DMA,
            pltpu.SemaphoreType.DMA,
            pltpu.SemaphoreType.DMA,
            pltpu.SemaphoreType.DMA,
        ],
    )
    def apply_kernel(d_hbm, lut_hbm, out_hbm, ibuf0, ibuf1, obuf0, obuf1,
                     lut, isem0, isem1, osem0, osem1):
        wid = lax.axis_index("s") * 2 + lax.axis_index("c")
        lane = lax.iota(jnp.int32, 16)
        ibufs = (ibuf0, ibuf1)
        obufs = (obuf0, obuf1)
        isems = (isem0, isem1)
        osems = (osem0, osem1)

        def lut_block(ib, ob):
            def one(x, sl):
                i = (x * 255.0).astype(jnp.int32)
                ob[sl] = plsc.load_gather(lut, [i * 16 + lane])

            _vec_loop(ib, _AROWS, 8, one)

        for r in range(3):
            t = wid * 3 + r
            p = t // 2
            h = t % 2
            bb = p // 3
            cc = p % 3
            r00 = h * 256
            pltpu.sync_copy(lut_hbm.at[pl.ds(p * 4096, 4096)], lut)

            def istart(ib_i, buf, sem):
                return pltpu.async_copy(
                    d_hbm.at[bb, cc, pl.ds(r00 + ib_i * _AROWS, _AROWS), :],
                    buf, sem)

            def ostart(ib_i, buf, sem):
                return pltpu.async_copy(
                    buf,
                    out_hbm.at[bb, cc, pl.ds(r00 + ib_i * _AROWS, _AROWS), :],
                    sem)

            icps = [None] * nab
            ocps = [None] * nab
            icps[0] = istart(0, ibuf0, isem0)
            icps[1] = istart(1, ibuf1, isem1)
            for ib in range(nab):
                bi = ib % 2
                icps[ib].wait()
                if ib >= 2:
                    ocps[ib - 2].wait()
                lut_block(ibufs[bi], obufs[bi])
                ocps[ib] = ostart(ib, obufs[bi], osems[bi])
                if ib + 2 < nab:
                    icps[ib + 2] = istart(ib + 2, ibufs[bi], isems[bi])
            ocps[nab - 2].wait()
            ocps[nab - 1].wait()

    return apply_kernel(dst, luts_flat)


_KMAP = np.array(
    [(lambda q: q if q < 16 else (q + 16) // 2)((p // 3) * (p % 3))
     for p in range(_NPLANE)],
    dtype=np.int32,
)


@jax.jit
def kernel(dst, ref):
    partials = _hist_sc(dst, ref).reshape(_NTASK, 256)
    luts24 = _tables_tc(partials)
    luts48 = luts24[_KMAP].reshape(-1)
    return _apply_sc(dst, luts48)
